# baseline (speedup 1.0000x reference)
import numpy as np
import jax
import jax.numpy as jnp
from jax.experimental import pallas as pl
from jax.experimental.pallas import tpu as pltpu

_TB = 256
_G = 64
_LN = 128


def _body(v_ref, vp_ref, wt_ref, lt_ref, bk_ref, wq_ref, ones_ref, o_ref,
          m_ref, den_ref, num_ref, s_scr):
    i = pl.program_id(1)
    tb = v_ref.shape[1]
    d = v_ref.shape[2]
    nchunk = d // _LN
    ng = tb // _G

    s_rep = jnp.dot(s_scr[...].astype(jnp.bfloat16), ones_ref[...],
                    preferred_element_type=jnp.float32)
    vp = vp_ref[0]

    @pl.when(i == 1)
    def _():
        m_ref[...] = jnp.full(m_ref.shape, -1e30, jnp.float32)
        den_ref[...] = jnp.zeros(den_ref.shape, jnp.float32)
        num_ref[...] = jnp.zeros(num_ref.shape, jnp.float32)

    m_old = m_ref[...]
    m_new = jnp.maximum(m_old, jnp.max(s_rep, axis=0, keepdims=True))
    alpha = jnp.exp(m_old - m_new)
    e_rep = jnp.exp(s_rep - m_new)
    e_bf = e_rep.astype(jnp.bfloat16)

    ev = jnp.concatenate(
        [vp[:, j * _LN:(j + 1) * _LN] * e_rep for j in range(nchunk)],
        axis=1).astype(jnp.bfloat16)
    nparts = []
    dparts = []
    for g in range(ng):
        rs = slice(g * _G, (g + 1) * _G)
        nparts.append(jnp.dot(lt_ref[...], ev[rs, :],
                              preferred_element_type=jnp.float32))
        dparts.append(jnp.dot(lt_ref[...], e_bf[rs, :],
                              preferred_element_type=jnp.float32))
    for g in range(1, ng):
        nparts[g] = nparts[g] + nparts[g - 1][_G - 1:_G, :]
        dparts[g] = dparts[g] + dparts[g - 1][_G - 1:_G, :]
    cums = jnp.concatenate(nparts, axis=0)
    dcum = jnp.concatenate(dparts, axis=0)

    den_full = den_ref[...] * alpha + dcum
    recip = 1.0 / den_full
    num_sc = jnp.concatenate(
        [num_ref[:, j * _LN:(j + 1) * _LN] * alpha for j in range(nchunk)],
        axis=1)
    num_full = num_sc + cums
    for j in range(nchunk):
        sl = slice(j * _LN, (j + 1) * _LN)
        o_ref[0, :, sl] = num_full[:, sl] * recip

    m_ref[...] = m_new
    den_ref[...] = den_full[tb - 1:tb, :]
    num_ref[...] = num_full[tb - 1:tb, :]

    v = v_ref[0]
    x = jnp.dot(v.astype(jnp.bfloat16), wt_ref[...],
                preferred_element_type=jnp.float32)
    k_act = jnp.tanh(x + bk_ref[...])
    s128 = k_act[:, 0:_LN] * wq_ref[0:1, :]
    for j in range(1, nchunk):
        s128 = s128 + k_act[:, j * _LN:(j + 1) * _LN] * wq_ref[j:j + 1, :]
    s_scr[...] = s128


def kernel(value, W_k, b_k, w_q):
    B, T, D = value.shape
    nt = T // _TB
    wt = W_k.T.astype(jnp.bfloat16)
    wq2 = w_q.reshape(D // _LN, _LN)
    ltri = jnp.asarray(np.tril(np.ones((_G, _G), np.float32)),
                       dtype=jnp.bfloat16)
    ones128 = jnp.ones((_LN, _LN), dtype=jnp.bfloat16)
    bk2 = b_k[None, :]
    return pl.pallas_call(
        _body,
        grid=(B, nt + 1),
        in_specs=[
            pl.BlockSpec((1, _TB, D),
                         lambda b, i: (b, jnp.minimum(i, nt - 1), 0)),
            pl.BlockSpec((1, _TB, D),
                         lambda b, i: (b, jnp.maximum(i - 1, 0), 0)),
            pl.BlockSpec((D, D), lambda b, i: (0, 0)),
            pl.BlockSpec((_G, _G), lambda b, i: (0, 0)),
            pl.BlockSpec((1, D), lambda b, i: (0, 0)),
            pl.BlockSpec((D // _LN, _LN), lambda b, i: (0, 0)),
            pl.BlockSpec((_LN, _LN), lambda b, i: (0, 0)),
        ],
        out_specs=pl.BlockSpec((1, _TB, D),
                               lambda b, i: (b, jnp.maximum(i - 1, 0), 0)),
        out_shape=jax.ShapeDtypeStruct((B, T, D), jnp.float32),
        scratch_shapes=[
            pltpu.VMEM((1, _LN), jnp.float32),
            pltpu.VMEM((1, _LN), jnp.float32),
            pltpu.VMEM((1, D), jnp.float32),
            pltpu.VMEM((_TB, _LN), jnp.float32),
        ],
        compiler_params=pltpu.CompilerParams(
            dimension_semantics=("parallel", "arbitrary"),
        ),
        name="step_attention_fused",
    )(value, value, wt, ltri, bk2, wq2, ones128)


# speedup vs baseline: 1.1272x; 1.1272x over previous
import numpy as np
import jax
import jax.numpy as jnp
from jax.experimental import pallas as pl
from jax.experimental.pallas import tpu as pltpu

_TB = 256
_G = 64
_LN = 128


def _body(v_ref, wt_ref, lt_ref, bk_ref, wq_ref, ones_ref, o_ref,
          m_ref, den_ref, num_ref, s_scr, v_scr):
    i = pl.program_id(1)
    tb = v_ref.shape[1]
    d = v_ref.shape[2]
    nchunk = d // _LN
    ng = tb // _G

    s_rep = jnp.dot(s_scr[...].astype(jnp.bfloat16), ones_ref[...],
                    preferred_element_type=jnp.float32)
    vp = v_scr[...]

    @pl.when(i == 1)
    def _():
        m_ref[...] = jnp.full(m_ref.shape, -1e30, jnp.float32)
        den_ref[...] = jnp.zeros(den_ref.shape, jnp.float32)
        num_ref[...] = jnp.zeros(num_ref.shape, jnp.float32)

    m_old = m_ref[...]
    m_new = jnp.maximum(m_old, jnp.max(s_rep, axis=0, keepdims=True))
    alpha = jnp.exp(m_old - m_new)
    e_rep = jnp.exp(s_rep - m_new)
    e_bf = e_rep.astype(jnp.bfloat16)

    ev = jnp.concatenate(
        [vp[:, j * _LN:(j + 1) * _LN] * e_rep for j in range(nchunk)],
        axis=1).astype(jnp.bfloat16)
    nparts = []
    dparts = []
    for g in range(ng):
        rs = slice(g * _G, (g + 1) * _G)
        nparts.append(jnp.dot(lt_ref[...], ev[rs, :],
                              preferred_element_type=jnp.float32))
        dparts.append(jnp.dot(lt_ref[...], e_bf[rs, :],
                              preferred_element_type=jnp.float32))
    for g in range(1, ng):
        nparts[g] = nparts[g] + nparts[g - 1][_G - 1:_G, :]
        dparts[g] = dparts[g] + dparts[g - 1][_G - 1:_G, :]
    cums = jnp.concatenate(nparts, axis=0)
    dcum = jnp.concatenate(dparts, axis=0)

    den_full = den_ref[...] * alpha + dcum
    recip = 1.0 / den_full
    num_sc = jnp.concatenate(
        [num_ref[:, j * _LN:(j + 1) * _LN] * alpha for j in range(nchunk)],
        axis=1)
    num_full = num_sc + cums
    for j in range(nchunk):
        sl = slice(j * _LN, (j + 1) * _LN)
        o_ref[0, :, sl] = num_full[:, sl] * recip

    m_ref[...] = m_new
    den_ref[...] = den_full[tb - 1:tb, :]
    num_ref[...] = num_full[tb - 1:tb, :]

    v = v_ref[0]
    x = jnp.dot(v.astype(jnp.bfloat16), wt_ref[...],
                preferred_element_type=jnp.float32)
    k_act = jnp.tanh(x + bk_ref[...])
    s128 = k_act[:, 0:_LN] * wq_ref[0:1, :]
    for j in range(1, nchunk):
        s128 = s128 + k_act[:, j * _LN:(j + 1) * _LN] * wq_ref[j:j + 1, :]
    s_scr[...] = s128
    v_scr[...] = v


def kernel(value, W_k, b_k, w_q):
    B, T, D = value.shape
    nt = T // _TB
    wt = W_k.T.astype(jnp.bfloat16)
    wq2 = w_q.reshape(D // _LN, _LN)
    ltri = jnp.asarray(np.tril(np.ones((_G, _G), np.float32)),
                       dtype=jnp.bfloat16)
    ones128 = jnp.ones((_LN, _LN), dtype=jnp.bfloat16)
    bk2 = b_k[None, :]
    return pl.pallas_call(
        _body,
        grid=(B, nt + 1),
        in_specs=[
            pl.BlockSpec((1, _TB, D),
                         lambda b, i: (b, jnp.minimum(i, nt - 1), 0)),
            pl.BlockSpec((D, D), lambda b, i: (0, 0)),
            pl.BlockSpec((_G, _G), lambda b, i: (0, 0)),
            pl.BlockSpec((1, D), lambda b, i: (0, 0)),
            pl.BlockSpec((D // _LN, _LN), lambda b, i: (0, 0)),
            pl.BlockSpec((_LN, _LN), lambda b, i: (0, 0)),
        ],
        out_specs=pl.BlockSpec((1, _TB, D),
                               lambda b, i: (b, jnp.maximum(i - 1, 0), 0)),
        out_shape=jax.ShapeDtypeStruct((B, T, D), jnp.float32),
        scratch_shapes=[
            pltpu.VMEM((1, _LN), jnp.float32),
            pltpu.VMEM((1, _LN), jnp.float32),
            pltpu.VMEM((1, D), jnp.float32),
            pltpu.VMEM((_TB, _LN), jnp.float32),
            pltpu.VMEM((_TB, D), jnp.float32),
        ],
        compiler_params=pltpu.CompilerParams(
            dimension_semantics=("parallel", "arbitrary"),
        ),
        name="step_attention_fused",
    )(value, wt, ltri, bk2, wq2, ones128)


# speedup vs baseline: 1.2600x; 1.1178x over previous
import numpy as np
import jax
import jax.numpy as jnp
from jax.experimental import pallas as pl
from jax.experimental.pallas import tpu as pltpu

_TB = 512
_G = 64
_LN = 128


def _body(v_ref, wt_ref, lt_ref, bk_ref, wq_ref, ones_ref, o_ref,
          m_ref, den_ref, num_ref, s_scr, v_scr):
    i = pl.program_id(1)
    tb = v_ref.shape[1]
    d = v_ref.shape[2]
    nchunk = d // _LN
    ng = tb // _G

    s_rep = jnp.dot(s_scr[...].astype(jnp.bfloat16), ones_ref[...],
                    preferred_element_type=jnp.float32)
    vp = v_scr[...]

    @pl.when(i == 1)
    def _():
        m_ref[...] = jnp.full(m_ref.shape, -1e30, jnp.float32)
        den_ref[...] = jnp.zeros(den_ref.shape, jnp.float32)
        num_ref[...] = jnp.zeros(num_ref.shape, jnp.float32)

    m_old = m_ref[...]
    m_new = jnp.maximum(m_old, jnp.max(s_rep, axis=0, keepdims=True))
    alpha = jnp.exp(m_old - m_new)
    e_rep = jnp.exp(s_rep - m_new)
    e_bf = e_rep.astype(jnp.bfloat16)

    ev = jnp.concatenate(
        [vp[:, j * _LN:(j + 1) * _LN] * e_rep for j in range(nchunk)],
        axis=1).astype(jnp.bfloat16)
    nparts = []
    dparts = []
    for g in range(ng):
        rs = slice(g * _G, (g + 1) * _G)
        nparts.append(jnp.dot(lt_ref[...], ev[rs, :],
                              preferred_element_type=jnp.float32))
        dparts.append(jnp.dot(lt_ref[...], e_bf[rs, :],
                              preferred_element_type=jnp.float32))
    npref = [nparts[0][_G - 1:_G, :]]
    dpref = [dparts[0][_G - 1:_G, :]]
    for g in range(1, ng - 1):
        npref.append(npref[-1] + nparts[g][_G - 1:_G, :])
        dpref.append(dpref[-1] + dparts[g][_G - 1:_G, :])
    for g in range(1, ng):
        nparts[g] = nparts[g] + npref[g - 1]
        dparts[g] = dparts[g] + dpref[g - 1]
    cums = jnp.concatenate(nparts, axis=0)
    dcum = jnp.concatenate(dparts, axis=0)

    den_full = den_ref[...] * alpha + dcum
    recip = 1.0 / den_full
    num_sc = jnp.concatenate(
        [num_ref[:, j * _LN:(j + 1) * _LN] * alpha for j in range(nchunk)],
        axis=1)
    num_full = num_sc + cums
    for j in range(nchunk):
        sl = slice(j * _LN, (j + 1) * _LN)
        o_ref[0, :, sl] = num_full[:, sl] * recip

    m_ref[...] = m_new
    den_ref[...] = den_full[tb - 1:tb, :]
    num_ref[...] = num_full[tb - 1:tb, :]

    v = v_ref[0]
    x = jnp.dot(v.astype(jnp.bfloat16), wt_ref[...],
                preferred_element_type=jnp.float32)
    k_act = jnp.tanh(x + bk_ref[...])
    s128 = k_act[:, 0:_LN] * wq_ref[0:1, :]
    for j in range(1, nchunk):
        s128 = s128 + k_act[:, j * _LN:(j + 1) * _LN] * wq_ref[j:j + 1, :]
    s_scr[...] = s128
    v_scr[...] = v


def kernel(value, W_k, b_k, w_q):
    B, T, D = value.shape
    nt = T // _TB
    wt = W_k.T.astype(jnp.bfloat16)
    wq2 = w_q.reshape(D // _LN, _LN)
    ltri = jnp.asarray(np.tril(np.ones((_G, _G), np.float32)),
                       dtype=jnp.bfloat16)
    ones128 = jnp.ones((_LN, _LN), dtype=jnp.bfloat16)
    bk2 = b_k[None, :]
    return pl.pallas_call(
        _body,
        grid=(B, nt + 1),
        in_specs=[
            pl.BlockSpec((1, _TB, D),
                         lambda b, i: (b, jnp.minimum(i, nt - 1), 0)),
            pl.BlockSpec((D, D), lambda b, i: (0, 0)),
            pl.BlockSpec((_G, _G), lambda b, i: (0, 0)),
            pl.BlockSpec((1, D), lambda b, i: (0, 0)),
            pl.BlockSpec((D // _LN, _LN), lambda b, i: (0, 0)),
            pl.BlockSpec((_LN, _LN), lambda b, i: (0, 0)),
        ],
        out_specs=pl.BlockSpec((1, _TB, D),
                               lambda b, i: (b, jnp.maximum(i - 1, 0), 0)),
        out_shape=jax.ShapeDtypeStruct((B, T, D), jnp.float32),
        scratch_shapes=[
            pltpu.VMEM((1, _LN), jnp.float32),
            pltpu.VMEM((1, _LN), jnp.float32),
            pltpu.VMEM((1, D), jnp.float32),
            pltpu.VMEM((_TB, _LN), jnp.float32),
            pltpu.VMEM((_TB, D), jnp.float32),
        ],
        compiler_params=pltpu.CompilerParams(
            dimension_semantics=("parallel", "arbitrary"),
        ),
        name="step_attention_fused",
    )(value, wt, ltri, bk2, wq2, ones128)


# speedup vs baseline: 1.3066x; 1.0370x over previous
import numpy as np
import jax
import jax.numpy as jnp
from jax.experimental import pallas as pl
from jax.experimental.pallas import tpu as pltpu

_TB = 512
_NB = 2
_G = 64
_LN = 128


def _stage_b(bb, i, vp, s128, lt_ref, ones_ref, o_ref, m_ref, den_ref,
             num_ref):
    tb, d = vp.shape
    nchunk = d // _LN
    ng = tb // _G

    s_rep = jnp.dot(s128.astype(jnp.bfloat16), ones_ref[...],
                    preferred_element_type=jnp.float32)

    m_old = m_ref[bb:bb + 1, :]
    m_new = jnp.maximum(m_old, jnp.max(s_rep, axis=0, keepdims=True))
    alpha = jnp.exp(m_old - m_new)
    e_rep = jnp.exp(s_rep - m_new)
    e_bf = e_rep.astype(jnp.bfloat16)

    ev = jnp.concatenate(
        [vp[:, j * _LN:(j + 1) * _LN] * e_rep for j in range(nchunk)],
        axis=1).astype(jnp.bfloat16)
    nparts = []
    dparts = []
    for g in range(ng):
        rs = slice(g * _G, (g + 1) * _G)
        nparts.append(jnp.dot(lt_ref[...], ev[rs, :],
                              preferred_element_type=jnp.float32))
        dparts.append(jnp.dot(lt_ref[...], e_bf[rs, :],
                              preferred_element_type=jnp.float32))
    npref = [nparts[0][_G - 1:_G, :]]
    dpref = [dparts[0][_G - 1:_G, :]]
    for g in range(1, ng - 1):
        npref.append(npref[-1] + nparts[g][_G - 1:_G, :])
        dpref.append(dpref[-1] + dparts[g][_G - 1:_G, :])
    for g in range(1, ng):
        nparts[g] = nparts[g] + npref[g - 1]
        dparts[g] = dparts[g] + dpref[g - 1]
    cums = jnp.concatenate(nparts, axis=0)
    dcum = jnp.concatenate(dparts, axis=0)

    den_full = den_ref[bb:bb + 1, :] * alpha + dcum
    recip = 1.0 / den_full
    num_sc = jnp.concatenate(
        [num_ref[bb:bb + 1, j * _LN:(j + 1) * _LN] * alpha
         for j in range(nchunk)], axis=1)
    num_full = num_sc + cums
    for j in range(nchunk):
        sl = slice(j * _LN, (j + 1) * _LN)
        o_ref[bb, :, sl] = num_full[:, sl] * recip

    m_ref[bb:bb + 1, :] = m_new
    den_ref[bb:bb + 1, :] = den_full[tb - 1:tb, :]
    num_ref[bb:bb + 1, :] = num_full[tb - 1:tb, :]


def _stage_a(bb, v_ref, wt_ref, bk_ref, wq_ref, s_scr, v_scr):
    d = v_ref.shape[2]
    nchunk = d // _LN
    v = v_ref[bb]
    x = jnp.dot(v.astype(jnp.bfloat16), wt_ref[...],
                preferred_element_type=jnp.float32)
    k_act = jnp.tanh(x + bk_ref[...])
    s128 = k_act[:, 0:_LN] * wq_ref[0:1, :]
    for j in range(1, nchunk):
        s128 = s128 + k_act[:, j * _LN:(j + 1) * _LN] * wq_ref[j:j + 1, :]
    s_scr[bb] = s128
    v_scr[bb] = v


def _body(v_ref, wt_ref, lt_ref, bk_ref, wq_ref, ones_ref, o_ref,
          m_ref, den_ref, num_ref, s_scr, v_scr):
    i = pl.program_id(1)

    parked = [(s_scr[bb], v_scr[bb]) for bb in range(_NB)]

    @pl.when(i == 1)
    def _():
        m_ref[...] = jnp.full(m_ref.shape, -1e30, jnp.float32)
        den_ref[...] = jnp.zeros(den_ref.shape, jnp.float32)
        num_ref[...] = jnp.zeros(num_ref.shape, jnp.float32)

    for bb in range(_NB):
        s128, vp = parked[bb][0], parked[bb][1]
        _stage_b(bb, i, vp, s128, lt_ref, ones_ref, o_ref, m_ref, den_ref,
                 num_ref)

    for bb in range(_NB):
        _stage_a(bb, v_ref, wt_ref, bk_ref, wq_ref, s_scr, v_scr)


def kernel(value, W_k, b_k, w_q):
    B, T, D = value.shape
    nt = T // _TB
    wt = W_k.T.astype(jnp.bfloat16)
    wq2 = w_q.reshape(D // _LN, _LN)
    ltri = jnp.asarray(np.tril(np.ones((_G, _G), np.float32)),
                       dtype=jnp.bfloat16)
    ones128 = jnp.ones((_LN, _LN), dtype=jnp.bfloat16)
    bk2 = b_k[None, :]
    return pl.pallas_call(
        _body,
        grid=(B // _NB, nt + 1),
        in_specs=[
            pl.BlockSpec((_NB, _TB, D),
                         lambda b, i: (b, jnp.minimum(i, nt - 1), 0)),
            pl.BlockSpec((D, D), lambda b, i: (0, 0)),
            pl.BlockSpec((_G, _G), lambda b, i: (0, 0)),
            pl.BlockSpec((1, D), lambda b, i: (0, 0)),
            pl.BlockSpec((D // _LN, _LN), lambda b, i: (0, 0)),
            pl.BlockSpec((_LN, _LN), lambda b, i: (0, 0)),
        ],
        out_specs=pl.BlockSpec((_NB, _TB, D),
                               lambda b, i: (b, jnp.maximum(i - 1, 0), 0)),
        out_shape=jax.ShapeDtypeStruct((B, T, D), jnp.float32),
        scratch_shapes=[
            pltpu.VMEM((_NB, _LN), jnp.float32),
            pltpu.VMEM((_NB, _LN), jnp.float32),
            pltpu.VMEM((_NB, D), jnp.float32),
            pltpu.VMEM((_NB, _TB, _LN), jnp.float32),
            pltpu.VMEM((_NB, _TB, D), jnp.float32),
        ],
        compiler_params=pltpu.CompilerParams(
            dimension_semantics=("parallel", "arbitrary"),
            vmem_limit_bytes=100 * 1024 * 1024,
        ),
        name="step_attention_fused",
    )(value, wt, ltri, bk2, wq2, ones128)


# speedup vs baseline: 1.3514x; 1.0342x over previous
import numpy as np
import jax
import jax.numpy as jnp
from jax.experimental import pallas as pl
from jax.experimental.pallas import tpu as pltpu

_TB = 512
_NB = 2
_G = 64
_LN = 128


def _stage_b(bb, i, vp, s128, lt_ref, ones_ref, o_ref, m_ref, den_ref,
             num_ref):
    tb, d = vp.shape
    nchunk = d // _LN
    ng = tb // _G

    s_rep = jnp.dot(s128.astype(jnp.bfloat16), ones_ref[...],
                    preferred_element_type=jnp.float32)

    m_old = m_ref[bb:bb + 1, :]
    m_new = jnp.maximum(m_old, jnp.max(s_rep, axis=0, keepdims=True))
    alpha = jnp.exp(m_old - m_new)
    e_rep = jnp.exp(s_rep - m_new)
    e_bf = e_rep.astype(jnp.bfloat16)

    ev = jnp.concatenate(
        [vp[:, j * _LN:(j + 1) * _LN] * e_bf for j in range(nchunk)],
        axis=1)
    nparts = []
    dparts = []
    for g in range(ng):
        rs = slice(g * _G, (g + 1) * _G)
        nparts.append(jnp.dot(lt_ref[...], ev[rs, :],
                              preferred_element_type=jnp.float32))
        dparts.append(jnp.dot(lt_ref[...], e_bf[rs, :],
                              preferred_element_type=jnp.float32))
    npref = [nparts[0][_G - 1:_G, :]]
    dpref = [dparts[0][_G - 1:_G, :]]
    for g in range(1, ng - 1):
        npref.append(npref[-1] + nparts[g][_G - 1:_G, :])
        dpref.append(dpref[-1] + dparts[g][_G - 1:_G, :])
    for g in range(1, ng):
        nparts[g] = nparts[g] + npref[g - 1]
        dparts[g] = dparts[g] + dpref[g - 1]
    cums = jnp.concatenate(nparts, axis=0)
    dcum = jnp.concatenate(dparts, axis=0)

    den_full = den_ref[bb:bb + 1, :] * alpha + dcum
    recip = 1.0 / den_full
    num_sc = jnp.concatenate(
        [num_ref[bb:bb + 1, j * _LN:(j + 1) * _LN] * alpha
         for j in range(nchunk)], axis=1)
    num_full = num_sc + cums
    for j in range(nchunk):
        sl = slice(j * _LN, (j + 1) * _LN)
        o_ref[bb, :, sl] = num_full[:, sl] * recip

    m_ref[bb:bb + 1, :] = m_new
    den_ref[bb:bb + 1, :] = den_full[tb - 1:tb, :]
    num_ref[bb:bb + 1, :] = num_full[tb - 1:tb, :]


def _stage_a(bb, v_ref, wt_ref, bk_ref, wq_ref, s_scr, v_scr):
    d = v_ref.shape[2]
    nchunk = d // _LN
    v = v_ref[bb]
    x = jnp.dot(v.astype(jnp.bfloat16), wt_ref[...],
                preferred_element_type=jnp.float32)
    k_act = jnp.tanh(x + bk_ref[...])
    s128 = k_act[:, 0:_LN] * wq_ref[0:1, :]
    for j in range(1, nchunk):
        s128 = s128 + k_act[:, j * _LN:(j + 1) * _LN] * wq_ref[j:j + 1, :]
    s_scr[bb] = s128
    v_scr[bb] = v.astype(jnp.bfloat16)


def _body(v_ref, wt_ref, lt_ref, bk_ref, wq_ref, ones_ref, o_ref,
          m_ref, den_ref, num_ref, s_scr, v_scr):
    i = pl.program_id(1)

    parked = [(s_scr[bb], v_scr[bb]) for bb in range(_NB)]

    @pl.when(i == 1)
    def _():
        m_ref[...] = jnp.full(m_ref.shape, -1e30, jnp.float32)
        den_ref[...] = jnp.zeros(den_ref.shape, jnp.float32)
        num_ref[...] = jnp.zeros(num_ref.shape, jnp.float32)

    for bb in range(_NB):
        s128, vp = parked[bb][0], parked[bb][1]
        _stage_b(bb, i, vp, s128, lt_ref, ones_ref, o_ref, m_ref, den_ref,
                 num_ref)

    for bb in range(_NB):
        _stage_a(bb, v_ref, wt_ref, bk_ref, wq_ref, s_scr, v_scr)


def kernel(value, W_k, b_k, w_q):
    B, T, D = value.shape
    nt = T // _TB
    wt = W_k.T.astype(jnp.bfloat16)
    wq2 = w_q.reshape(D // _LN, _LN)
    ltri = jnp.asarray(np.tril(np.ones((_G, _G), np.float32)),
                       dtype=jnp.bfloat16)
    ones128 = jnp.ones((_LN, _LN), dtype=jnp.bfloat16)
    bk2 = b_k[None, :]
    return pl.pallas_call(
        _body,
        grid=(B // _NB, nt + 1),
        in_specs=[
            pl.BlockSpec((_NB, _TB, D),
                         lambda b, i: (b, jnp.minimum(i, nt - 1), 0)),
            pl.BlockSpec((D, D), lambda b, i: (0, 0)),
            pl.BlockSpec((_G, _G), lambda b, i: (0, 0)),
            pl.BlockSpec((1, D), lambda b, i: (0, 0)),
            pl.BlockSpec((D // _LN, _LN), lambda b, i: (0, 0)),
            pl.BlockSpec((_LN, _LN), lambda b, i: (0, 0)),
        ],
        out_specs=pl.BlockSpec((_NB, _TB, D),
                               lambda b, i: (b, jnp.maximum(i - 1, 0), 0)),
        out_shape=jax.ShapeDtypeStruct((B, T, D), jnp.float32),
        scratch_shapes=[
            pltpu.VMEM((_NB, _LN), jnp.float32),
            pltpu.VMEM((_NB, _LN), jnp.float32),
            pltpu.VMEM((_NB, D), jnp.float32),
            pltpu.VMEM((_NB, _TB, _LN), jnp.float32),
            pltpu.VMEM((_NB, _TB, D), jnp.bfloat16),
        ],
        compiler_params=pltpu.CompilerParams(
            dimension_semantics=("parallel", "arbitrary"),
            vmem_limit_bytes=100 * 1024 * 1024,
        ),
        name="step_attention_fused",
    )(value, wt, ltri, bk2, wq2, ones128)


# speedup vs baseline: 1.3961x; 1.0331x over previous
import numpy as np
import jax
import jax.numpy as jnp
from jax.experimental import pallas as pl
from jax.experimental.pallas import tpu as pltpu

_TB = 512
_NB = 2
_G = 64
_LN = 128


def _body(v_ref, wt_ref, lt_ref, bk_ref, wq_ref, ones_ref, o_ref,
          m_ref, den_ref, num_ref, s_scr, v_scr):
    i = pl.program_id(1)
    tb = v_ref.shape[1]
    d = v_ref.shape[2]
    nchunk = d // _LN
    ng = tb // _G

    parked_s = [s_scr[bb] for bb in range(_NB)]
    parked_v = [v_scr[bb] for bb in range(_NB)]

    @pl.when(i == 1)
    def _():
        m_ref[...] = jnp.full(m_ref.shape, -1e30, jnp.float32)
        den_ref[...] = jnp.zeros(den_ref.shape, jnp.float32)
        num_ref[...] = jnp.zeros(num_ref.shape, jnp.float32)

    e_bfs = []
    alphas = []
    m_news = []
    for bb in range(_NB):
        s_rep = jnp.dot(parked_s[bb].astype(jnp.bfloat16), ones_ref[...],
                        preferred_element_type=jnp.float32)
        m_old = m_ref[bb:bb + 1, :]
        m_new = jnp.maximum(m_old, jnp.max(s_rep, axis=0, keepdims=True))
        alphas.append(jnp.exp(m_old - m_new))
        m_news.append(m_new)
        e_bfs.append(jnp.exp(s_rep - m_new).astype(jnp.bfloat16))

    ev = jnp.concatenate(
        [parked_v[bb][:, j * _LN:(j + 1) * _LN] * e_bfs[bb]
         for bb in range(_NB) for j in range(nchunk)], axis=1)
    e_cat = jnp.concatenate(e_bfs, axis=1)

    nparts = []
    dparts = []
    for g in range(ng):
        rs = slice(g * _G, (g + 1) * _G)
        nparts.append(jnp.dot(lt_ref[...], ev[rs, :],
                              preferred_element_type=jnp.float32))
        dparts.append(jnp.dot(lt_ref[...], e_cat[rs, :],
                              preferred_element_type=jnp.float32))
    npref = [nparts[0][_G - 1:_G, :]]
    dpref = [dparts[0][_G - 1:_G, :]]
    for g in range(1, ng - 1):
        npref.append(npref[-1] + nparts[g][_G - 1:_G, :])
        dpref.append(dpref[-1] + dparts[g][_G - 1:_G, :])
    for g in range(1, ng):
        nparts[g] = nparts[g] + npref[g - 1]
        dparts[g] = dparts[g] + dpref[g - 1]
    cums = jnp.concatenate(nparts, axis=0)
    dcum = jnp.concatenate(dparts, axis=0)

    for bb in range(_NB):
        den_full = (den_ref[bb:bb + 1, :] * alphas[bb]
                    + dcum[:, bb * _LN:(bb + 1) * _LN])
        recip = 1.0 / den_full
        num_sc = jnp.concatenate(
            [num_ref[bb:bb + 1, j * _LN:(j + 1) * _LN] * alphas[bb]
             for j in range(nchunk)], axis=1)
        num_full = num_sc + cums[:, bb * d:(bb + 1) * d]
        for j in range(nchunk):
            sl = slice(j * _LN, (j + 1) * _LN)
            o_ref[bb, :, sl] = num_full[:, sl] * recip
        m_ref[bb:bb + 1, :] = m_news[bb]
        den_ref[bb:bb + 1, :] = den_full[tb - 1:tb, :]
        num_ref[bb:bb + 1, :] = num_full[tb - 1:tb, :]

    vv = v_ref[...].reshape(_NB * tb, d)
    vv_bf = vv.astype(jnp.bfloat16)
    x = jnp.dot(vv_bf, wt_ref[...],
                preferred_element_type=jnp.float32)
    k_act = jnp.tanh(x + bk_ref[...])
    s128 = k_act[:, 0:_LN] * wq_ref[0:1, :]
    for j in range(1, nchunk):
        s128 = s128 + k_act[:, j * _LN:(j + 1) * _LN] * wq_ref[j:j + 1, :]
    s_scr[...] = s128.reshape(_NB, tb, _LN)
    v_scr[...] = vv_bf.reshape(_NB, tb, d)


def kernel(value, W_k, b_k, w_q):
    B, T, D = value.shape
    nt = T // _TB
    wt = W_k.T.astype(jnp.bfloat16)
    wq2 = w_q.reshape(D // _LN, _LN)
    ltri = jnp.asarray(np.tril(np.ones((_G, _G), np.float32)),
                       dtype=jnp.bfloat16)
    ones128 = jnp.ones((_LN, _LN), dtype=jnp.bfloat16)
    bk2 = b_k[None, :]
    return pl.pallas_call(
        _body,
        grid=(B // _NB, nt + 1),
        in_specs=[
            pl.BlockSpec((_NB, _TB, D),
                         lambda b, i: (b, jnp.minimum(i, nt - 1), 0)),
            pl.BlockSpec((D, D), lambda b, i: (0, 0)),
            pl.BlockSpec((_G, _G), lambda b, i: (0, 0)),
            pl.BlockSpec((1, D), lambda b, i: (0, 0)),
            pl.BlockSpec((D // _LN, _LN), lambda b, i: (0, 0)),
            pl.BlockSpec((_LN, _LN), lambda b, i: (0, 0)),
        ],
        out_specs=pl.BlockSpec((_NB, _TB, D),
                               lambda b, i: (b, jnp.maximum(i - 1, 0), 0)),
        out_shape=jax.ShapeDtypeStruct((B, T, D), jnp.float32),
        scratch_shapes=[
            pltpu.VMEM((_NB, _LN), jnp.float32),
            pltpu.VMEM((_NB, _LN), jnp.float32),
            pltpu.VMEM((_NB, D), jnp.float32),
            pltpu.VMEM((_NB, _TB, _LN), jnp.float32),
            pltpu.VMEM((_NB, _TB, D), jnp.bfloat16),
        ],
        compiler_params=pltpu.CompilerParams(
            dimension_semantics=("parallel", "arbitrary"),
            vmem_limit_bytes=100 * 1024 * 1024,
        ),
        name="step_attention_fused",
    )(value, wt, ltri, bk2, wq2, ones128)
